# Initial kernel scaffold; baseline (speedup 1.0000x reference)
import jax
import jax.numpy as jnp
from jax.experimental import pallas as pl


def kernel(e, edge_index, l):
    raise NotImplementedError("write your pallas kernel here")


# speedup vs baseline: 276.3421x; 276.3421x over previous
import functools

import jax
import jax.numpy as jnp
from jax import lax
from jax.experimental import pallas as pl
from jax.experimental.pallas import tpu as pltpu
from jax.experimental.pallas import tpu_sc as plsc

N_NODES = 100000
N_EDGES = 6400000

NC = 2
NS = 16
NW = NC * NS
EPW = N_EDGES // NW
CHUNK = 2000
NCHUNK = EPW // CHUNK
LANES = 16

NPAD = 100352
NSLICE = NPAD // NS


def _zero_vmem(ref, nwords):
    def body(i, c):
        ref[pl.ds(i * LANES, LANES)] = jnp.zeros((LANES,), jnp.float32)
        return c
    lax.fori_loop(0, nwords // LANES, body, 0)


def _make_edge_pass(with_cnt):
    mesh = plsc.VectorSubcoreMesh(core_axis_name="c", subcore_axis_name="s")

    out_type = [jax.ShapeDtypeStruct((NC, NPAD), jnp.float32)]
    if with_cnt:
        out_type.append(jax.ShapeDtypeStruct((NC, NPAD), jnp.float32))

    scratch = [
        pltpu.VMEM((N_NODES,), jnp.float32),
        pltpu.VMEM((CHUNK,), jnp.int32),
        pltpu.VMEM((CHUNK,), jnp.int32),
        pltpu.VMEM((CHUNK,), jnp.float32),
        pltpu.VMEM((CHUNK,), jnp.float32),
        pltpu.VMEM((NSLICE,), jnp.float32),
        pltpu.VMEM_SHARED((NPAD,), jnp.float32),
        pltpu.VMEM_SHARED((NPAD,), jnp.float32),
    ]

    @functools.partial(
        pl.kernel,
        mesh=mesh,
        out_type=out_type,
        scratch_types=scratch,
        compiler_params=pltpu.CompilerParams(needs_layout_passes=False),
    )
    def edge_pass(e_hbm, row_hbm, col_hbm, *refs):
        if with_cnt:
            sums_hbm, cnt_hbm = refs[0], refs[1]
            refs = refs[2:]
        else:
            sums_hbm = refs[0]
            refs = refs[1:]
        e_v, row_v, col_v, msg_v, ones_v, z_v, sums_s, cnt_s = refs

        cid = lax.axis_index("c")
        sid = lax.axis_index("s")
        wid = sid * NC + cid

        _zero_vmem(z_v, NSLICE)
        pltpu.sync_copy(z_v, sums_s.at[pl.ds(sid * NSLICE, NSLICE)])
        if with_cnt:
            pltpu.sync_copy(z_v, cnt_s.at[pl.ds(sid * NSLICE, NSLICE)])

        if with_cnt:
            def ones_body(i, c):
                ones_v[pl.ds(i * LANES, LANES)] = jnp.ones((LANES,), jnp.float32)
                return c
            lax.fori_loop(0, CHUNK // LANES, ones_body, 0)

        pltpu.sync_copy(e_hbm, e_v)

        plsc.subcore_barrier()

        base0 = wid * EPW

        def chunk_body(ci, carry):
            base = base0 + ci * CHUNK
            pltpu.sync_copy(row_hbm.at[pl.ds(base, CHUNK)], row_v)
            pltpu.sync_copy(col_hbm.at[pl.ds(base, CHUNK)], col_v)

            def gather_body(g, c):
                idx = row_v[pl.ds(g * LANES, LANES)]
                msg_v[pl.ds(g * LANES, LANES)] = plsc.load_gather(e_v, [idx])
                return c
            lax.fori_loop(0, CHUNK // LANES, gather_body, 0)

            pltpu.sync_copy(msg_v, sums_s.at[col_v], add=True)
            if with_cnt:
                pltpu.sync_copy(ones_v, cnt_s.at[col_v], add=True)
            return carry

        lax.fori_loop(0, NCHUNK, chunk_body, 0)

        plsc.subcore_barrier()

        sl = pl.ds(sid * NSLICE, NSLICE)
        pltpu.sync_copy(sums_s.at[sl], sums_hbm.at[cid, sl])
        if with_cnt:
            pltpu.sync_copy(cnt_s.at[sl], cnt_hbm.at[cid, sl])

    return edge_pass


_edge_pass_first = _make_edge_pass(True)
_edge_pass_next = _make_edge_pass(False)

_R = NPAD // 128


def _update_body(e_ref, s_ref, c_ref, o_ref):
    agg = s_ref[0] + s_ref[1]
    deg = c_ref[0] + c_ref[1]
    o_ref[...] = e_ref[...] * 0.5 + 0.5 * agg / jnp.maximum(deg, 1.0)


def _update(e_pad, sums, cnt):
    out = pl.pallas_call(
        _update_body,
        out_shape=jax.ShapeDtypeStruct((_R, 128), jnp.float32),
    )(e_pad.reshape(_R, 128), sums.reshape(NC, _R, 128),
      cnt.reshape(NC, _R, 128))
    return out.reshape(NPAD)


def _as_tuple(r):
    return tuple(r) if isinstance(r, (list, tuple)) else (r,)


def kernel(e, edge_index, l):
    row = edge_index[0]
    col = edge_index[1]
    e_pad = jnp.pad(e, (0, NPAD - N_NODES))

    def body(i, carry):
        e_p, cnt = carry
        e_cur = e_p[:N_NODES]

        def first(_):
            s, c = _as_tuple(_edge_pass_first(e_cur, row, col))
            return s, c

        def later(_):
            (s,) = _as_tuple(_edge_pass_next(e_cur, row, col))
            return s, cnt

        sums, cnt2 = lax.cond(i == 0, first, later, None)
        return _update(e_p, sums, cnt2), cnt2

    cnt0 = jnp.zeros((NC, NPAD), jnp.float32)
    e_out, _ = lax.fori_loop(0, l, body, (e_pad, cnt0))
    return e_out[:N_NODES]


# speedup vs baseline: 568.5752x; 2.0575x over previous
import functools

import jax
import jax.numpy as jnp
from jax import lax
from jax.experimental import pallas as pl
from jax.experimental.pallas import tpu as pltpu
from jax.experimental.pallas import tpu_sc as plsc

N_NODES = 100000
N_EDGES = 6400000

NC = 2
NS = 16
NW = NC * NS
EPW = N_EDGES // NW
CHUNK = 800
NCHUNK = EPW // CHUNK
LANES = 16
NBUF = 5
LOOK = 2
NOUTER = NCHUNK // NBUF
GUNROLL = 5


NPAD = 100352
NSLICE = NPAD // NS
ZCH = 784


def _make_edge_pass(with_cnt):
    mesh = plsc.VectorSubcoreMesh(core_axis_name="c", subcore_axis_name="s")

    out_type = [jax.ShapeDtypeStruct((NC, NPAD), jnp.float32)]
    if with_cnt:
        out_type.append(jax.ShapeDtypeStruct((NC, NPAD), jnp.float32))

    scratch = [
        pltpu.VMEM((N_NODES,), jnp.float32),
        [pltpu.VMEM((CHUNK,), jnp.int32) for _ in range(NBUF)],
        [pltpu.VMEM((CHUNK,), jnp.int32) for _ in range(NBUF)],
        [pltpu.VMEM((CHUNK,), jnp.float32) for _ in range(NBUF)],
        pltpu.VMEM((CHUNK,), jnp.float32),
        pltpu.VMEM_SHARED((NPAD,), jnp.float32),
        pltpu.VMEM_SHARED((NPAD,), jnp.float32),
        [pltpu.SemaphoreType.DMA for _ in range(NBUF)],
        [pltpu.SemaphoreType.DMA for _ in range(NBUF)],
        [pltpu.SemaphoreType.DMA for _ in range(NBUF)],
    ]

    @functools.partial(
        pl.kernel,
        mesh=mesh,
        out_type=out_type,
        scratch_types=scratch,
        compiler_params=pltpu.CompilerParams(needs_layout_passes=False),
    )
    def edge_pass(e_hbm, row_hbm, col_hbm, *refs):
        if with_cnt:
            sums_hbm, cnt_hbm = refs[0], refs[1]
            refs = refs[2:]
        else:
            sums_hbm = refs[0]
            refs = refs[1:]
        (e_v, row_v, col_v, msg_v, ones_v, sums_s, cnt_s,
         in_sem, s_sem, c_sem) = refs

        cid = lax.axis_index("c")
        sid = lax.axis_index("s")
        wid = sid * NC + cid
        base0 = wid * EPW

        def zinit(i, c):
            msg_v[0][pl.ds(i * LANES, LANES)] = jnp.zeros((LANES,), jnp.float32)
            return c
        lax.fori_loop(0, CHUNK // LANES, zinit, 0)
        for k in range(NSLICE // ZCH):
            dst = pl.ds(sid * NSLICE + k * ZCH, ZCH)
            pltpu.sync_copy(msg_v[0].at[pl.ds(0, ZCH)], sums_s.at[dst])
            if with_cnt:
                pltpu.sync_copy(msg_v[0].at[pl.ds(0, ZCH)], cnt_s.at[dst])

        if with_cnt:
            def ones_body(i, c):
                ones_v[pl.ds(i * LANES, LANES)] = jnp.ones((LANES,), jnp.float32)
                return c
            lax.fori_loop(0, CHUNK // LANES, ones_body, 0)

        pltpu.sync_copy(e_hbm, e_v)

        plsc.subcore_barrier()

        def issue_in(ci, b):
            src = pl.ds(base0 + ci * CHUNK, CHUNK)
            pltpu.async_copy(row_hbm.at[src], row_v[b], in_sem[b])
            pltpu.async_copy(col_hbm.at[src], col_v[b], in_sem[b])

        def wait_in(b):
            pltpu.make_async_copy(row_hbm.at[pl.ds(0, CHUNK)], row_v[b],
                                  in_sem[b]).wait()
            pltpu.make_async_copy(col_hbm.at[pl.ds(0, CHUNK)], col_v[b],
                                  in_sem[b]).wait()

        def drain_scatter(b):
            pltpu.make_async_copy(msg_v[b], sums_s.at[col_v[b]],
                                  s_sem[b]).wait()
            if with_cnt:
                pltpu.make_async_copy(ones_v, cnt_s.at[col_v[b]],
                                      c_sem[b]).wait()

        for ci in range(LOOK):
            issue_in(ci, ci)

        LAG = NBUF - LOOK

        def outer(j, carry):
            for b in range(NBUF):
                ci = j * NBUF + b
                wait_in(b)

                def gather(g, c):
                    for u in range(GUNROLL):
                        off = (g * GUNROLL + u) * LANES
                        idx = row_v[b][pl.ds(off, LANES)]
                        msg_v[b][pl.ds(off, LANES)] = plsc.load_gather(
                            e_v, [idx])
                    return c
                lax.fori_loop(0, CHUNK // (LANES * GUNROLL), gather, 0)

                pltpu.async_copy(msg_v[b], sums_s.at[col_v[b]], s_sem[b],
                                 add=True)
                if with_cnt:
                    pltpu.async_copy(ones_v, cnt_s.at[col_v[b]], c_sem[b],
                                     add=True)

                bn = (b + LOOK) % NBUF
                if b < LAG:
                    @pl.when(ci >= LAG)
                    def _():
                        drain_scatter(bn)
                    issue_in(ci + LOOK, bn)
                else:
                    drain_scatter(bn)

                    @pl.when(ci + LOOK < NCHUNK)
                    def _():
                        issue_in(ci + LOOK, bn)
            return carry

        lax.fori_loop(0, NOUTER, outer, 0)

        for b in range(LOOK, NBUF):
            drain_scatter(b)

        plsc.subcore_barrier()

        sl = pl.ds(sid * NSLICE, NSLICE)
        pltpu.sync_copy(sums_s.at[sl], sums_hbm.at[cid, sl])
        if with_cnt:
            pltpu.sync_copy(cnt_s.at[sl], cnt_hbm.at[cid, sl])

    return edge_pass


_edge_pass_first = _make_edge_pass(True)
_edge_pass_next = _make_edge_pass(False)

_R = NPAD // 128


def _update_body(e_ref, s_ref, c_ref, o_ref):
    agg = s_ref[0] + s_ref[1]
    deg = c_ref[0] + c_ref[1]
    o_ref[...] = e_ref[...] * 0.5 + 0.5 * agg / jnp.maximum(deg, 1.0)


def _update(e_pad, sums, cnt):
    out = pl.pallas_call(
        _update_body,
        out_shape=jax.ShapeDtypeStruct((_R, 128), jnp.float32),
    )(e_pad.reshape(_R, 128), sums.reshape(NC, _R, 128),
      cnt.reshape(NC, _R, 128))
    return out.reshape(NPAD)


def _as_tuple(r):
    return tuple(r) if isinstance(r, (list, tuple)) else (r,)


def kernel(e, edge_index, l):
    row = edge_index[0]
    col = edge_index[1]
    e_pad = jnp.pad(e, (0, NPAD - N_NODES))

    def body(i, carry):
        e_p, cnt = carry
        e_cur = e_p[:N_NODES]

        def first(_):
            s, c = _as_tuple(_edge_pass_first(e_cur, row, col))
            return s, c

        def later(_):
            (s,) = _as_tuple(_edge_pass_next(e_cur, row, col))
            return s, cnt

        sums, cnt2 = lax.cond(i == 0, first, later, None)
        return _update(e_p, sums, cnt2), cnt2

    cnt0 = jnp.zeros((NC, NPAD), jnp.float32)
    e_out, _ = lax.fori_loop(0, l, body, (e_pad, cnt0))
    return e_out[:N_NODES]
